# baseline (speedup 1.0000x reference)
import functools

import jax
import jax.numpy as jnp
from jax.experimental import pallas as pl
from jax.experimental.pallas import tpu as pltpu

B, CH, H, W = 2, 128, 64, 64
N = H * W
SPARSITY = 0.8
K_KEEP = int(N * SPARSITY)
BQ = 256
INT32_MIN = -2147483648


def _row_count_ge(key, cand):
    return jnp.sum((key >= cand).astype(jnp.float32), axis=1, keepdims=True)


def _attn_kernel(xt_ref, WqT_ref, bq_ref, Wk_ref, bk2_ref,
                 WvT_ref, bv_ref, out_ref, kt_s, v_s):
    j = pl.program_id(1)

    @pl.when(j == 0)
    def _():
        kt_s[...] = (jax.lax.dot_general(
            Wk_ref[...], xt_ref[0], (((1,), (1,)), ((), ())),
            preferred_element_type=jnp.float32) + bk2_ref[...])
        v_s[:, :CH] = (jnp.dot(xt_ref[0], WvT_ref[...],
                               preferred_element_type=jnp.float32)
                       + bv_ref[...])
        v_s[:, CH:] = jnp.ones((N, 8), jnp.float32)

    q_blk = xt_ref[0, pl.ds(j * BQ, BQ), :]
    q = jnp.dot(q_blk, WqT_ref[...], preferred_element_type=jnp.float32) + bq_ref[...]
    s = jnp.dot(q, kt_s[...], preferred_element_type=jnp.float32) * (1.0 / (CH ** 0.5))

    bits = jax.lax.bitcast_convert_type(s, jnp.int32)
    key = bits ^ (jax.lax.shift_right_arithmetic(bits, 31) & 0x7FFFFFFF)

    HB = BQ // 2
    ones_col = jnp.ones((N, 8), jnp.bfloat16)
    keys = [key[:HB], key[HB:]]

    def count(kh, cand):
        mask = (kh >= cand).astype(jnp.bfloat16)
        return jnp.dot(mask, ones_col, preferred_element_type=jnp.float32)[:, 0:1]

    prefixes = []
    for kh in keys:
        cnt0 = count(kh, jnp.zeros((HB, 1), jnp.int32))
        prefixes.append(jnp.where(cnt0 >= K_KEEP, 0, INT32_MIN).astype(jnp.int32))

    for bit in range(30, -1, -1):
        cands = [prefixes[h] | (1 << bit) for h in (0, 1)]
        cnts = [count(keys[h], cands[h]) for h in (0, 1)]
        prefixes = [jnp.where(cnts[h] >= K_KEEP, cands[h], prefixes[h])
                    for h in (0, 1)]
    thr = jnp.concatenate(prefixes, axis=0)
    keep = key >= thr

    rowmax = jnp.max(s, axis=1, keepdims=True)
    p = jnp.where(keep, jnp.exp(s - rowmax), 0.0)
    out_aug = jnp.dot(p, v_s[...], preferred_element_type=jnp.float32)
    out_ref[0] = out_aug[:, :CH] / out_aug[:, CH:CH + 1]


_OFF1 = tuple((dh, dw) for dh in (-1, 0, 1) for dw in (-1, 0, 1))
_OFF2 = tuple((dh, dw) for dh in (-2, 0, 2) for dw in (-2, 0, 2))


def _shift_flat(x, dh, dw, wcol):
    d = dh * W + dw
    c = x.shape[0]
    if d > 0:
        xs = jnp.concatenate([x[:, d:], jnp.zeros((c, d), x.dtype)], axis=1)
    elif d < 0:
        xs = jnp.concatenate([jnp.zeros((c, -d), x.dtype), x[:, :N + d]], axis=1)
    else:
        xs = x
    if dw > 0:
        xs = xs * (wcol <= (W - 1 - dw)).astype(x.dtype)
    elif dw < 0:
        xs = xs * (wcol >= (-dw)).astype(x.dtype)
    return xs


def _tap_sum(w9_ref, xin, offsets, wcol):
    acc = None
    for t, (dh, dw) in enumerate(offsets):
        xs = _shift_flat(xin, dh, dw, wcol)
        term = jnp.dot(w9_ref[t], xs, preferred_element_type=jnp.float32)
        acc = term if acc is None else acc + term
    return acc


def _conv_kernel(x2_ref, Wg9_ref, bg_ref, Wr9_ref, br2_ref, Wf9_ref, bf2_ref,
                 out_ref):
    x2 = x2_ref[0]
    wcol = jax.lax.broadcasted_iota(jnp.int32, (1, N), 1) % W

    g_l = _tap_sum(Wg9_ref, x2, _OFF1, wcol)
    g = jax.nn.sigmoid(g_l[0:1, :] + bg_ref[0, 0])
    xi = x2 * g
    xr = _tap_sum(Wr9_ref, xi, _OFF1, wcol) + br2_ref[...]
    x3 = x2 + xr
    y = _tap_sum(Wf9_ref, x3, _OFF2, wcol) + bf2_ref[...]
    out_ref[0] = y


@functools.partial(jax.jit, static_argnames=("interpret",))
def _run(x, Wq, bq, Wk, bk, Wv, bv, Wr, br, Wg, bg, Wf, bf, interpret=False):
    Bsz, C, Hh, Ww = x.shape
    xt = x.reshape(Bsz, N, C)

    nj = N // BQ
    attn_out = pl.pallas_call(
        _attn_kernel,
        grid=(Bsz, nj),
        in_specs=[
            pl.BlockSpec((1, N, C), lambda b, j: (b, 0, 0)),
            pl.BlockSpec((C, C), lambda b, j: (0, 0)),
            pl.BlockSpec((1, C), lambda b, j: (0, 0)),
            pl.BlockSpec((C, C), lambda b, j: (0, 0)),
            pl.BlockSpec((C, 1), lambda b, j: (0, 0)),
            pl.BlockSpec((C, C), lambda b, j: (0, 0)),
            pl.BlockSpec((1, C), lambda b, j: (0, 0)),
        ],
        out_specs=pl.BlockSpec((1, BQ, C), lambda b, j: (b, j, 0)),
        out_shape=jax.ShapeDtypeStruct((Bsz, N, C), jnp.float32),
        scratch_shapes=[
            pltpu.VMEM((C, N), jnp.float32),
            pltpu.VMEM((N, C + 8), jnp.float32),
        ],
        interpret=interpret,
    )(xt, Wq.T, bq.reshape(1, C), Wk, bk.reshape(C, 1),
      Wv.T, bv.reshape(1, C))

    x2 = attn_out.reshape(Bsz, C, N)
    Wg9 = jnp.pad(Wg.transpose(2, 3, 0, 1).reshape(9, 1, C), ((0, 0), (0, 7), (0, 0)))
    Wr9 = Wr.transpose(2, 3, 0, 1).reshape(9, C, C)
    Wf9 = Wf.transpose(2, 3, 0, 1).reshape(9, C, C)

    y = pl.pallas_call(
        _conv_kernel,
        grid=(Bsz,),
        in_specs=[
            pl.BlockSpec((1, C, N), lambda b: (b, 0, 0)),
            pl.BlockSpec((9, 8, C), lambda b: (0, 0, 0)),
            pl.BlockSpec((1, 1), lambda b: (0, 0)),
            pl.BlockSpec((9, C, C), lambda b: (0, 0, 0)),
            pl.BlockSpec((C, 1), lambda b: (0, 0)),
            pl.BlockSpec((9, C, C), lambda b: (0, 0, 0)),
            pl.BlockSpec((C, 1), lambda b: (0, 0)),
        ],
        out_specs=pl.BlockSpec((1, C, N), lambda b: (b, 0, 0)),
        out_shape=jax.ShapeDtypeStruct((Bsz, C, N), jnp.float32),
        interpret=interpret,
    )(x2, Wg9, bg.reshape(1, 1), Wr9, br.reshape(C, 1), Wf9, bf.reshape(C, 1))

    return y.reshape(Bsz, C, Hh, Ww)


def kernel(x, Wq, bq, Wk, bk, Wv, bv, Wr, br, Wg, bg, Wf, bf):
    return _run(x, Wq, bq, Wk, bk, Wv, bv, Wr, br, Wg, bg, Wf, bf)


# speedup vs baseline: 1.2587x; 1.2587x over previous
import functools

import jax
import jax.numpy as jnp
from jax.experimental import pallas as pl
from jax.experimental.pallas import tpu as pltpu

B, CH, H, W = 2, 128, 64, 64
N = H * W
SPARSITY = 0.8
K_KEEP = int(N * SPARSITY)
BQ = 256
INT32_MIN = -2147483648


def _row_count_ge(key, cand):
    return jnp.sum((key >= cand).astype(jnp.float32), axis=1, keepdims=True)


def _attn_kernel(xt_ref, WqT_ref, bq_ref, Wk_ref, bk2_ref,
                 WvT_ref, bv_ref, out_ref, kt_s, v_s):
    j = pl.program_id(1)

    @pl.when(j == 0)
    def _():
        kt_s[...] = (jax.lax.dot_general(
            Wk_ref[...], xt_ref[0], (((1,), (1,)), ((), ())),
            preferred_element_type=jnp.float32) + bk2_ref[...])
        v_s[:, :CH] = (jnp.dot(xt_ref[0], WvT_ref[...],
                               preferred_element_type=jnp.float32)
                       + bv_ref[...])
        v_s[:, CH:] = jnp.ones((N, 8), jnp.float32)

    q_blk = xt_ref[0, pl.ds(j * BQ, BQ), :]
    q = jnp.dot(q_blk, WqT_ref[...], preferred_element_type=jnp.float32) + bq_ref[...]
    s = jnp.dot(q, kt_s[...], preferred_element_type=jnp.float32) * (1.0 / (CH ** 0.5))

    bits = jax.lax.bitcast_convert_type(s, jnp.int32)
    key = bits ^ (jax.lax.shift_right_arithmetic(bits, 31) & 0x7FFFFFFF)

    HB = BQ // 2
    keys = [key[:HB], key[HB:]]
    prefixes = []
    for kh in keys:
        cnt0 = jnp.sum((kh >= 0).astype(jnp.float32), axis=1, keepdims=True)
        prefixes.append(jnp.where(cnt0 >= K_KEEP, 0, INT32_MIN).astype(jnp.int32))

    for bit in range(30, -1, -1):
        cands = [prefixes[h] | (1 << bit) for h in (0, 1)]
        cnts = [jnp.sum((keys[h] >= cands[h]).astype(jnp.float32),
                        axis=1, keepdims=True) for h in (0, 1)]
        prefixes = [jnp.where(cnts[h] >= K_KEEP, cands[h], prefixes[h])
                    for h in (0, 1)]
    thr = jnp.concatenate(prefixes, axis=0)
    keep = key >= thr

    rowmax = jnp.max(s, axis=1, keepdims=True)
    p = jnp.where(keep, jnp.exp(s - rowmax), 0.0)
    out_aug = jnp.dot(p, v_s[...], preferred_element_type=jnp.float32)
    out_ref[0] = out_aug[:, :CH] / out_aug[:, CH:CH + 1]


_OFF1 = tuple((dh, dw) for dh in (-1, 0, 1) for dw in (-1, 0, 1))
_OFF2 = tuple((dh, dw) for dh in (-2, 0, 2) for dw in (-2, 0, 2))


def _shift_flat(x, dh, dw, wcol):
    d = dh * W + dw
    c = x.shape[0]
    if d > 0:
        xs = jnp.concatenate([x[:, d:], jnp.zeros((c, d), x.dtype)], axis=1)
    elif d < 0:
        xs = jnp.concatenate([jnp.zeros((c, -d), x.dtype), x[:, :N + d]], axis=1)
    else:
        xs = x
    if dw > 0:
        xs = xs * (wcol <= (W - 1 - dw)).astype(x.dtype)
    elif dw < 0:
        xs = xs * (wcol >= (-dw)).astype(x.dtype)
    return xs


def _tap_sum(w9_ref, xin, offsets, wcol):
    acc = None
    for t, (dh, dw) in enumerate(offsets):
        xs = _shift_flat(xin, dh, dw, wcol)
        term = jnp.dot(w9_ref[t], xs, preferred_element_type=jnp.float32)
        acc = term if acc is None else acc + term
    return acc


def _conv_kernel(x2_ref, Wg9_ref, bg_ref, Wr9_ref, br2_ref, Wf9_ref, bf2_ref,
                 out_ref):
    x2 = x2_ref[0]
    wcol = jax.lax.broadcasted_iota(jnp.int32, (1, N), 1) % W

    g_l = _tap_sum(Wg9_ref, x2, _OFF1, wcol)
    g = jax.nn.sigmoid(g_l[0:1, :] + bg_ref[0, 0])
    xi = x2 * g
    xr = _tap_sum(Wr9_ref, xi, _OFF1, wcol) + br2_ref[...]
    x3 = x2 + xr
    y = _tap_sum(Wf9_ref, x3, _OFF2, wcol) + bf2_ref[...]
    out_ref[0] = y


@functools.partial(jax.jit, static_argnames=("interpret",))
def _run(x, Wq, bq, Wk, bk, Wv, bv, Wr, br, Wg, bg, Wf, bf, interpret=False):
    Bsz, C, Hh, Ww = x.shape
    xt = x.reshape(Bsz, N, C)

    nj = N // BQ
    attn_out = pl.pallas_call(
        _attn_kernel,
        grid=(Bsz, nj),
        in_specs=[
            pl.BlockSpec((1, N, C), lambda b, j: (b, 0, 0)),
            pl.BlockSpec((C, C), lambda b, j: (0, 0)),
            pl.BlockSpec((1, C), lambda b, j: (0, 0)),
            pl.BlockSpec((C, C), lambda b, j: (0, 0)),
            pl.BlockSpec((C, 1), lambda b, j: (0, 0)),
            pl.BlockSpec((C, C), lambda b, j: (0, 0)),
            pl.BlockSpec((1, C), lambda b, j: (0, 0)),
        ],
        out_specs=pl.BlockSpec((1, BQ, C), lambda b, j: (b, j, 0)),
        out_shape=jax.ShapeDtypeStruct((Bsz, N, C), jnp.float32),
        scratch_shapes=[
            pltpu.VMEM((C, N), jnp.float32),
            pltpu.VMEM((N, C + 8), jnp.float32),
        ],
        interpret=interpret,
    )(xt, Wq.T, bq.reshape(1, C), Wk, bk.reshape(C, 1),
      Wv.T, bv.reshape(1, C))

    x2 = attn_out.reshape(Bsz, C, N)
    Wg9 = jnp.pad(Wg.transpose(2, 3, 0, 1).reshape(9, 1, C), ((0, 0), (0, 7), (0, 0)))
    Wr9 = Wr.transpose(2, 3, 0, 1).reshape(9, C, C)
    Wf9 = Wf.transpose(2, 3, 0, 1).reshape(9, C, C)

    y = pl.pallas_call(
        _conv_kernel,
        grid=(Bsz,),
        in_specs=[
            pl.BlockSpec((1, C, N), lambda b: (b, 0, 0)),
            pl.BlockSpec((9, 8, C), lambda b: (0, 0, 0)),
            pl.BlockSpec((1, 1), lambda b: (0, 0)),
            pl.BlockSpec((9, C, C), lambda b: (0, 0, 0)),
            pl.BlockSpec((C, 1), lambda b: (0, 0)),
            pl.BlockSpec((9, C, C), lambda b: (0, 0, 0)),
            pl.BlockSpec((C, 1), lambda b: (0, 0)),
        ],
        out_specs=pl.BlockSpec((1, C, N), lambda b: (b, 0, 0)),
        out_shape=jax.ShapeDtypeStruct((Bsz, C, N), jnp.float32),
        interpret=interpret,
    )(x2, Wg9, bg.reshape(1, 1), Wr9, br.reshape(C, 1), Wf9, bf.reshape(C, 1))

    return y.reshape(Bsz, C, Hh, Ww)


def kernel(x, Wq, bq, Wk, bk, Wv, bv, Wr, br, Wg, bg, Wf, bf):
    return _run(x, Wq, bq, Wk, bk, Wv, bv, Wr, br, Wg, bg, Wf, bf)
